# baseline (speedup 1.0000x reference)
import functools
import math

import jax
import jax.numpy as jnp
from jax import lax
from jax.experimental import pallas as pl
from jax.experimental.pallas import tpu as pltpu
from jax.experimental.pallas import tpu_sc as plsc

B, S, D = 2, 2048, 1024
H = 16
E = 8
DH = D // H

SBLK = 512
SQ = 1024
NCH = 4
HQ = SQ // NCH
RB = 256
SB = 512

_CONTRACT_LAST = (((1,), (1,)), ((), ()))
_CONTRACT_STD = (((1,), (0,)), ((), ()))


def _qkv_body(x_ref, wq_ref, bq_ref, wk_ref, bk_ref, wv_ref, bv_ref,
              q_ref, k_ref, v_ref):
    x = x_ref[0].astype(jnp.bfloat16)
    q = (lax.dot_general(x, wq_ref[...], _CONTRACT_LAST,
                         preferred_element_type=jnp.float32)
         + bq_ref[...]).astype(jnp.bfloat16)
    k = (lax.dot_general(x, wk_ref[...], _CONTRACT_LAST,
                         preferred_element_type=jnp.float32)
         + bk_ref[...]).astype(jnp.bfloat16)
    v = (lax.dot_general(x, wv_ref[...], _CONTRACT_LAST,
                         preferred_element_type=jnp.float32)
         + bv_ref[...]).astype(jnp.bfloat16)
    for h in range(H):
        sl = slice(h * DH, (h + 1) * DH)
        q_ref[0, h] = q[:, sl]
        k_ref[0, h] = k[:, sl]
        v_ref[0, h] = v[:, sl]


def _attn_body(q_ref, k_ref, v_ref, ctx_ref, mean_ref):
    qi = pl.program_id(2)

    k = k_ref[0, 0]
    v = v_ref[0, 0]

    part = jnp.zeros((1, DH), jnp.float32)
    for half in range(NCH):
        q = q_ref[0, 0, half * HQ:(half + 1) * HQ]
        scores = lax.dot_general(q, k, _CONTRACT_LAST,
                                 preferred_element_type=jnp.float32)
        m = jnp.max(scores, axis=1, keepdims=True)
        p = jnp.exp2(scores - m)
        l = jnp.sum(p, axis=1, keepdims=True)
        ctx = lax.dot_general(p.astype(jnp.bfloat16), v, _CONTRACT_STD,
                              preferred_element_type=jnp.float32) / l
        ctx_ref[0, 0, half * HQ:(half + 1) * HQ] = ctx.astype(jnp.bfloat16)
        part = part + jnp.sum(ctx, axis=0, keepdims=True)

    @pl.when(qi == 0)
    def _():
        mean_ref[...] = jnp.zeros_like(mean_ref)

    mean_ref[0, 0, 0:1, :] += part * (1.0 / S)


def _logits_body(g_ref, wg_ref, bg_ref, logits_ref):
    g = g_ref[...]
    logits = lax.dot_general(g, wg_ref[...], _CONTRACT_LAST,
                             preferred_element_type=jnp.float32) + bg_ref[...]
    pad = jnp.full((B, 16 - E), -1e30, jnp.float32)
    logits_ref[...] = jnp.concatenate([logits, pad], axis=1)


def _sc_gate_body(logits_hbm, w_hbm, buf, out_buf):
    c = lax.axis_index("c")
    s_ = lax.axis_index("s")

    idx = lax.iota(jnp.int32, 16)

    gdn = lax.GatherDimensionNumbers(offset_dims=(), collapsed_slice_dims=(0,),
                                     start_index_map=(0,))

    def lane_gather(vec, perm):
        return lax.gather(vec, perm.reshape(16, 1), gdn, (1,),
                          mode=lax.GatherScatterMode.PROMISE_IN_BOUNDS)

    def allred(vec, op):
        for sh in (8, 4, 2, 1):
            perm = jnp.bitwise_xor(idx, sh)
            vec = op(vec, lane_gather(vec, perm))
        return vec

    @pl.when(jnp.logical_and(c == 0, s_ == 0))
    def _():
        pltpu.sync_copy(logits_hbm, buf)
        for b in range(B):
            x = buf[b]
            m = allred(x, jnp.maximum)
            p = jnp.exp(x - m)
            p = p / allred(p, jnp.add)
            v1 = allred(p, jnp.maximum)
            i1 = allred(jnp.where(p >= v1, idx, 16), jnp.minimum)
            mask1 = idx == i1
            p2 = jnp.where(mask1, -1.0, p)
            v2 = allred(p2, jnp.maximum)
            i2 = allred(jnp.where(p2 >= v2, idx, 16), jnp.minimum)
            out_buf[b] = (jnp.where(mask1, v1, 0.0)
                          + jnp.where(idx == i2, v2, 0.0))
        pltpu.sync_copy(out_buf, w_hbm)


def _combine_body(w_ref, we_ref, be_ref, wc_ref, bc_ref):
    we = we_ref[...].astype(jnp.float32)
    for b in range(B):
        acc = w_ref[b, 0] * we[0]
        for e in range(1, E):
            acc = acc + w_ref[b, e] * we[e]
        wc_ref[b] = acc.astype(jnp.bfloat16)

    @pl.when(pl.program_id(0) == 0)
    def _():
        be = be_ref[...]
        for b in range(B):
            bcc = w_ref[b, 0] * be[0:1]
            for e in range(1, E):
                bcc = bcc + w_ref[b, e] * be[e:e + 1]
            bc_ref[b] = bcc


def _moe_out_body(ctx_ref, wc_ref, bc_ref, wo_ref, bo_ref, out_ref):
    ctx = jnp.concatenate([ctx_ref[0, h] for h in range(H)], axis=1)
    moe = lax.dot_general(ctx, wc_ref[0], _CONTRACT_LAST,
                          preferred_element_type=jnp.float32) + bc_ref[0]
    out = lax.dot_general(moe.astype(jnp.bfloat16), wo_ref[...], _CONTRACT_LAST,
                          preferred_element_type=jnp.float32) + bo_ref[...]
    out_ref[0] = out


def kernel(hidden_states, Wq, bq, Wk, bk, Wv, bv, We, be, Wg, bg, Wo, bo):
    scale = math.log2(math.e) / math.sqrt(DH)
    wq16 = (Wq * scale).astype(jnp.bfloat16)
    wk16 = Wk.astype(jnp.bfloat16)
    wv16 = Wv.astype(jnp.bfloat16)
    we16 = We.astype(jnp.bfloat16)
    wo16 = Wo.astype(jnp.bfloat16)
    bq2 = (bq * scale).reshape(1, D)
    bk2 = bk.reshape(1, D)
    bv2 = bv.reshape(1, D)
    bg2 = bg.reshape(1, E)
    bo2 = bo.reshape(1, D)

    n_sb = S // SBLK
    qkv_struct = jax.ShapeDtypeStruct((B, H, S, DH), jnp.bfloat16)
    q, k, v = pl.pallas_call(
        _qkv_body,
        grid=(B, n_sb),
        in_specs=[
            pl.BlockSpec((1, SBLK, D), lambda b, si: (b, si, 0)),
            pl.BlockSpec((D, D), lambda b, si: (0, 0)),
            pl.BlockSpec((1, D), lambda b, si: (0, 0)),
            pl.BlockSpec((D, D), lambda b, si: (0, 0)),
            pl.BlockSpec((1, D), lambda b, si: (0, 0)),
            pl.BlockSpec((D, D), lambda b, si: (0, 0)),
            pl.BlockSpec((1, D), lambda b, si: (0, 0)),
        ],
        out_specs=[
            pl.BlockSpec((1, H, SBLK, DH), lambda b, si: (b, 0, si, 0)),
            pl.BlockSpec((1, H, SBLK, DH), lambda b, si: (b, 0, si, 0)),
            pl.BlockSpec((1, H, SBLK, DH), lambda b, si: (b, 0, si, 0)),
        ],
        out_shape=[qkv_struct] * 3,
    )(hidden_states, wq16, bq2, wk16, bk2, wv16, bv2)

    nq = S // SQ
    ctx4, means = pl.pallas_call(
        _attn_body,
        grid=(B, H, nq),
        in_specs=[
            pl.BlockSpec((1, 1, SQ, DH), lambda b, h, qi: (b, h, qi, 0)),
            pl.BlockSpec((1, 1, S, DH), lambda b, h, qi: (b, h, 0, 0)),
            pl.BlockSpec((1, 1, S, DH), lambda b, h, qi: (b, h, 0, 0)),
        ],
        out_specs=[
            pl.BlockSpec((1, 1, SQ, DH), lambda b, h, qi: (b, h, qi, 0)),
            pl.BlockSpec((1, 1, 8, DH), lambda b, h, qi: (b, h, 0, 0)),
        ],
        out_shape=[
            jax.ShapeDtypeStruct((B, H, S, DH), jnp.bfloat16),
            jax.ShapeDtypeStruct((B, H, 8, DH), jnp.float32),
        ],
    )(q, k, v)

    gate_input = means[:, :, 0, :].reshape(B, D)

    logits16 = pl.pallas_call(
        _logits_body,
        grid=(1,),
        in_specs=[
            pl.BlockSpec((B, D), lambda i: (0, 0)),
            pl.BlockSpec((E, D), lambda i: (0, 0)),
            pl.BlockSpec((1, E), lambda i: (0, 0)),
        ],
        out_specs=pl.BlockSpec((B, 16), lambda i: (0, 0)),
        out_shape=jax.ShapeDtypeStruct((B, 16), jnp.float32),
    )(gate_input, Wg, bg2)

    sc_gate = pl.kernel(
        _sc_gate_body,
        out_type=jax.ShapeDtypeStruct((B, 16), jnp.float32),
        mesh=plsc.VectorSubcoreMesh(core_axis_name="c", subcore_axis_name="s"),
        scratch_types=[
            pltpu.VMEM((B, 16), jnp.float32),
            pltpu.VMEM((B, 16), jnp.float32),
        ],
    )
    w = sc_gate(logits16)[:, :E]

    n_rb = D // RB
    wc, bc3 = pl.pallas_call(
        _combine_body,
        grid=(n_rb,),
        in_specs=[
            pl.BlockSpec(memory_space=pltpu.SMEM),
            pl.BlockSpec((E, RB, D), lambda i: (0, i, 0)),
            pl.BlockSpec((E, D), lambda i: (0, 0)),
        ],
        out_specs=[
            pl.BlockSpec((B, RB, D), lambda i: (0, i, 0)),
            pl.BlockSpec((B, 1, D), lambda i: (0, 0, 0)),
        ],
        out_shape=[
            jax.ShapeDtypeStruct((B, D, D), jnp.bfloat16),
            jax.ShapeDtypeStruct((B, 1, D), jnp.float32),
        ],
    )(w, we16, be)
    ns = S // SB
    out = pl.pallas_call(
        _moe_out_body,
        grid=(B, ns),
        in_specs=[
            pl.BlockSpec((1, H, SB, DH), lambda b, si: (b, 0, si, 0)),
            pl.BlockSpec((1, D, D), lambda b, si: (b, 0, 0)),
            pl.BlockSpec((1, 1, D), lambda b, si: (b, 0, 0)),
            pl.BlockSpec((D, D), lambda b, si: (0, 0)),
            pl.BlockSpec((1, D), lambda b, si: (0, 0)),
        ],
        out_specs=pl.BlockSpec((1, SB, D), lambda b, si: (b, si, 0)),
        out_shape=jax.ShapeDtypeStruct((B, S, D), jnp.float32),
    )(ctx4, wc, bc3, wo16, bo2)

    return out


# speedup vs baseline: 1.0480x; 1.0480x over previous
import functools
import math

import jax
import jax.numpy as jnp
from jax import lax
from jax.experimental import pallas as pl
from jax.experimental.pallas import tpu as pltpu
from jax.experimental.pallas import tpu_sc as plsc

B, S, D = 2, 2048, 1024
H = 16
E = 8
DH = D // H

SBLK = 512
SQ = 1024
NCH = 4
HQ = SQ // NCH
RB = 256
SB = 512

_CONTRACT_LAST = (((1,), (1,)), ((), ()))
_CONTRACT_STD = (((1,), (0,)), ((), ()))


def _qkv_body(x_ref, wq_ref, bq_ref, wk_ref, bk_ref, wv_ref, bv_ref,
              q_ref, k_ref, v_ref):
    x = x_ref[0].astype(jnp.bfloat16)
    q = (lax.dot_general(x, wq_ref[...], _CONTRACT_LAST,
                         preferred_element_type=jnp.float32)
         + bq_ref[...]).astype(jnp.bfloat16)
    k = (lax.dot_general(x, wk_ref[...], _CONTRACT_LAST,
                         preferred_element_type=jnp.float32)
         + bk_ref[...]).astype(jnp.bfloat16)
    v = (lax.dot_general(x, wv_ref[...], _CONTRACT_LAST,
                         preferred_element_type=jnp.float32)
         + bv_ref[...]).astype(jnp.bfloat16)
    for h in range(H):
        sl = slice(h * DH, (h + 1) * DH)
        q_ref[0, h] = q[:, sl]
        k_ref[0, h] = k[:, sl]
        v_ref[0, h] = v[:, sl]


def _attn_body(q_ref, k_ref, v_ref, wg_ref, bg_ref, ctx_ref, mean_ref,
               logits_ref):
    b_ = pl.program_id(0)
    h_ = pl.program_id(1)
    qi = pl.program_id(2)

    k = k_ref[0, 0]
    v = v_ref[0, 0]
    v_aug = jnp.concatenate([v, jnp.ones((S, 1), jnp.bfloat16)], axis=1)

    part = jnp.zeros((1, DH), jnp.float32)
    for half in range(NCH):
        q = q_ref[0, 0, half * HQ:(half + 1) * HQ]
        scores = lax.dot_general(q, k, _CONTRACT_LAST,
                                 preferred_element_type=jnp.float32)
        m = jnp.max(scores, axis=1, keepdims=True)
        p = jnp.exp2(scores - m)
        ctx_aug = lax.dot_general(p.astype(jnp.bfloat16), v_aug, _CONTRACT_STD,
                                  preferred_element_type=jnp.float32)
        ctx = ctx_aug[:, :DH] / ctx_aug[:, DH:DH + 1]
        ctx_ref[0, 0, half * HQ:(half + 1) * HQ] = ctx.astype(jnp.bfloat16)
        part = part + jnp.sum(ctx, axis=0, keepdims=True)

    @pl.when(jnp.logical_and(b_ == 0, jnp.logical_and(h_ == 0, qi == 0)))
    def _():
        mean_ref[...] = jnp.zeros_like(mean_ref)

    mean_ref[pl.ds(b_, 1), pl.ds(h_, 1), 0:1, :] += part.reshape(
        1, 1, 1, DH) * (1.0 / S)

    is_last = jnp.logical_and(
        b_ == B - 1, jnp.logical_and(h_ == H - 1, qi == pl.num_programs(2) - 1))

    @pl.when(is_last)
    def _():
        rows = []
        for b in range(B):
            rows.append(jnp.concatenate(
                [mean_ref[b, h, 0:1, :] for h in range(H)], axis=1))
        g = jnp.concatenate(rows, axis=0)
        logits = lax.dot_general(g, wg_ref[...], _CONTRACT_LAST,
                                 preferred_element_type=jnp.float32) + bg_ref[...]
        pad = jnp.full((B, 16 - E), -1e30, jnp.float32)
        logits_ref[...] = jnp.concatenate([logits, pad], axis=1)


def _sc_gate_body(logits_hbm, w_hbm, buf, out_buf):
    c = lax.axis_index("c")
    s_ = lax.axis_index("s")

    idx = lax.iota(jnp.int32, 16)

    gdn = lax.GatherDimensionNumbers(offset_dims=(), collapsed_slice_dims=(0,),
                                     start_index_map=(0,))

    def lane_gather(vec, perm):
        return lax.gather(vec, perm.reshape(16, 1), gdn, (1,),
                          mode=lax.GatherScatterMode.PROMISE_IN_BOUNDS)

    def allred(vec, op):
        for sh in (8, 4, 2, 1):
            perm = jnp.bitwise_xor(idx, sh)
            vec = op(vec, lane_gather(vec, perm))
        return vec

    @pl.when(jnp.logical_and(c == 0, s_ == 0))
    def _():
        pltpu.sync_copy(logits_hbm, buf)
        for b in range(B):
            x = buf[b]
            m = allred(x, jnp.maximum)
            p = jnp.exp(x - m)
            p = p / allred(p, jnp.add)
            v1 = allred(p, jnp.maximum)
            i1 = allred(jnp.where(p >= v1, idx, 16), jnp.minimum)
            mask1 = idx == i1
            p2 = jnp.where(mask1, -1.0, p)
            v2 = allred(p2, jnp.maximum)
            i2 = allred(jnp.where(p2 >= v2, idx, 16), jnp.minimum)
            out_buf[b] = (jnp.where(mask1, v1, 0.0)
                          + jnp.where(idx == i2, v2, 0.0))
        pltpu.sync_copy(out_buf, w_hbm)


def _combine_body(w_ref, we_ref, be_ref, wc_ref, bc_ref):
    we = we_ref[...].astype(jnp.float32)
    for b in range(B):
        acc = w_ref[b, 0] * we[0]
        for e in range(1, E):
            acc = acc + w_ref[b, e] * we[e]
        wc_ref[b] = acc.astype(jnp.bfloat16)

    @pl.when(pl.program_id(0) == 0)
    def _():
        be = be_ref[...]
        for b in range(B):
            bcc = w_ref[b, 0] * be[0:1]
            for e in range(1, E):
                bcc = bcc + w_ref[b, e] * be[e:e + 1]
            bc_ref[b] = bcc


def _moe_out_body(ctx_ref, wc_ref, bc_ref, wo_ref, bo_ref, out_ref):
    ctx = jnp.concatenate([ctx_ref[0, h] for h in range(H)], axis=1)
    moe = lax.dot_general(ctx, wc_ref[0], _CONTRACT_LAST,
                          preferred_element_type=jnp.float32) + bc_ref[0]
    out = lax.dot_general(moe.astype(jnp.bfloat16), wo_ref[...], _CONTRACT_LAST,
                          preferred_element_type=jnp.float32) + bo_ref[...]
    out_ref[0] = out


def kernel(hidden_states, Wq, bq, Wk, bk, Wv, bv, We, be, Wg, bg, Wo, bo):
    scale = math.log2(math.e) / math.sqrt(DH)
    wq16 = (Wq * scale).astype(jnp.bfloat16)
    wk16 = Wk.astype(jnp.bfloat16)
    wv16 = Wv.astype(jnp.bfloat16)
    we16 = We.astype(jnp.bfloat16)
    wo16 = Wo.astype(jnp.bfloat16)
    bq2 = (bq * scale).reshape(1, D)
    bk2 = bk.reshape(1, D)
    bv2 = bv.reshape(1, D)
    bg2 = bg.reshape(1, E)
    bo2 = bo.reshape(1, D)

    n_sb = S // SBLK
    qkv_struct = jax.ShapeDtypeStruct((B, H, S, DH), jnp.bfloat16)
    q, k, v = pl.pallas_call(
        _qkv_body,
        grid=(B, n_sb),
        in_specs=[
            pl.BlockSpec((1, SBLK, D), lambda b, si: (b, si, 0)),
            pl.BlockSpec((D, D), lambda b, si: (0, 0)),
            pl.BlockSpec((1, D), lambda b, si: (0, 0)),
            pl.BlockSpec((D, D), lambda b, si: (0, 0)),
            pl.BlockSpec((1, D), lambda b, si: (0, 0)),
            pl.BlockSpec((D, D), lambda b, si: (0, 0)),
            pl.BlockSpec((1, D), lambda b, si: (0, 0)),
        ],
        out_specs=[
            pl.BlockSpec((1, H, SBLK, DH), lambda b, si: (b, 0, si, 0)),
            pl.BlockSpec((1, H, SBLK, DH), lambda b, si: (b, 0, si, 0)),
            pl.BlockSpec((1, H, SBLK, DH), lambda b, si: (b, 0, si, 0)),
        ],
        out_shape=[qkv_struct] * 3,
    )(hidden_states, wq16, bq2, wk16, bk2, wv16, bv2)

    nq = S // SQ
    ctx4, _means, logits16 = pl.pallas_call(
        _attn_body,
        grid=(B, H, nq),
        in_specs=[
            pl.BlockSpec((1, 1, SQ, DH), lambda b, h, qi: (b, h, qi, 0)),
            pl.BlockSpec((1, 1, S, DH), lambda b, h, qi: (b, h, 0, 0)),
            pl.BlockSpec((1, 1, S, DH), lambda b, h, qi: (b, h, 0, 0)),
            pl.BlockSpec((E, D), lambda b, h, qi: (0, 0)),
            pl.BlockSpec((1, E), lambda b, h, qi: (0, 0)),
        ],
        out_specs=[
            pl.BlockSpec((1, 1, SQ, DH), lambda b, h, qi: (b, h, qi, 0)),
            pl.BlockSpec((B, H, 8, DH), lambda b, h, qi: (0, 0, 0, 0)),
            pl.BlockSpec((B, 16), lambda b, h, qi: (0, 0)),
        ],
        out_shape=[
            jax.ShapeDtypeStruct((B, H, S, DH), jnp.bfloat16),
            jax.ShapeDtypeStruct((B, H, 8, DH), jnp.float32),
            jax.ShapeDtypeStruct((B, 16), jnp.float32),
        ],
    )(q, k, v, Wg, bg2)

    sc_gate = pl.kernel(
        _sc_gate_body,
        out_type=jax.ShapeDtypeStruct((B, 16), jnp.float32),
        mesh=plsc.VectorSubcoreMesh(core_axis_name="c", subcore_axis_name="s"),
        scratch_types=[
            pltpu.VMEM((B, 16), jnp.float32),
            pltpu.VMEM((B, 16), jnp.float32),
        ],
    )
    w = sc_gate(logits16)[:, :E]

    n_rb = D // RB
    wc, bc3 = pl.pallas_call(
        _combine_body,
        grid=(n_rb,),
        in_specs=[
            pl.BlockSpec(memory_space=pltpu.SMEM),
            pl.BlockSpec((E, RB, D), lambda i: (0, i, 0)),
            pl.BlockSpec((E, D), lambda i: (0, 0)),
        ],
        out_specs=[
            pl.BlockSpec((B, RB, D), lambda i: (0, i, 0)),
            pl.BlockSpec((B, 1, D), lambda i: (0, 0, 0)),
        ],
        out_shape=[
            jax.ShapeDtypeStruct((B, D, D), jnp.bfloat16),
            jax.ShapeDtypeStruct((B, 1, D), jnp.float32),
        ],
    )(w, we16, be)
    ns = S // SB
    out = pl.pallas_call(
        _moe_out_body,
        grid=(B, ns),
        in_specs=[
            pl.BlockSpec((1, H, SB, DH), lambda b, si: (b, 0, si, 0)),
            pl.BlockSpec((1, D, D), lambda b, si: (b, 0, 0)),
            pl.BlockSpec((1, 1, D), lambda b, si: (b, 0, 0)),
            pl.BlockSpec((D, D), lambda b, si: (0, 0)),
            pl.BlockSpec((1, D), lambda b, si: (0, 0)),
        ],
        out_specs=pl.BlockSpec((1, SB, D), lambda b, si: (b, si, 0)),
        out_shape=jax.ShapeDtypeStruct((B, S, D), jnp.float32),
    )(ctx4, wc, bc3, wo16, bo2)

    return out


# speedup vs baseline: 1.1480x; 1.0955x over previous
import functools
import math

import jax
import jax.numpy as jnp
from jax import lax
from jax.experimental import pallas as pl
from jax.experimental.pallas import tpu as pltpu
from jax.experimental.pallas import tpu_sc as plsc

B, S, D = 2, 2048, 1024
H = 16
E = 8
DH = D // H

SBLK = 1024
SQ = 2048
NCH = 4
HQ = SQ // NCH
RB = 256
SB = 1024

_CONTRACT_LAST = (((1,), (1,)), ((), ()))
_CONTRACT_STD = (((1,), (0,)), ((), ()))


def _qkv_body(x_ref, wq_ref, bq_ref, wk_ref, bk_ref, wv_ref, bv_ref,
              q_ref, k_ref, v_ref):
    x = x_ref[0].astype(jnp.bfloat16)
    q = (lax.dot_general(x, wq_ref[...], _CONTRACT_LAST,
                         preferred_element_type=jnp.float32)
         + bq_ref[...]).astype(jnp.bfloat16)
    k = (lax.dot_general(x, wk_ref[...], _CONTRACT_LAST,
                         preferred_element_type=jnp.float32)
         + bk_ref[...]).astype(jnp.bfloat16)
    v = (lax.dot_general(x, wv_ref[...], _CONTRACT_LAST,
                         preferred_element_type=jnp.float32)
         + bv_ref[...]).astype(jnp.bfloat16)
    for h in range(H):
        sl = slice(h * DH, (h + 1) * DH)
        q_ref[0, h] = q[:, sl]
        k_ref[0, h] = k[:, sl]
        v_ref[0, h] = v[:, sl]


def _attn_body(q_ref, k_ref, v_ref, wg_ref, bg_ref, ctx_ref, mean_ref,
               logits_ref):
    b_ = pl.program_id(0)
    h_ = pl.program_id(1)
    qi = pl.program_id(2)

    k = k_ref[0, 0]
    v = v_ref[0, 0]
    v_aug = jnp.concatenate([v, jnp.ones((S, 1), jnp.bfloat16)], axis=1)

    part = jnp.zeros((1, DH), jnp.float32)
    for half in range(NCH):
        q = q_ref[0, 0, half * HQ:(half + 1) * HQ]
        scores = lax.dot_general(q, k, _CONTRACT_LAST,
                                 preferred_element_type=jnp.float32)
        m = jnp.max(scores, axis=1, keepdims=True)
        p = jnp.exp2(scores - m)
        ctx_aug = lax.dot_general(p.astype(jnp.bfloat16), v_aug, _CONTRACT_STD,
                                  preferred_element_type=jnp.float32)
        ctx = ctx_aug[:, :DH] / ctx_aug[:, DH:DH + 1]
        ctx_ref[0, 0, half * HQ:(half + 1) * HQ] = ctx.astype(jnp.bfloat16)
        part = part + jnp.sum(ctx, axis=0, keepdims=True)

    @pl.when(jnp.logical_and(b_ == 0, jnp.logical_and(h_ == 0, qi == 0)))
    def _():
        mean_ref[...] = jnp.zeros_like(mean_ref)

    mean_ref[pl.ds(b_, 1), pl.ds(h_, 1), 0:1, :] += part.reshape(
        1, 1, 1, DH) * (1.0 / S)

    is_last = jnp.logical_and(
        b_ == B - 1, jnp.logical_and(h_ == H - 1, qi == pl.num_programs(2) - 1))

    @pl.when(is_last)
    def _():
        rows = []
        for b in range(B):
            rows.append(jnp.concatenate(
                [mean_ref[b, h, 0:1, :] for h in range(H)], axis=1))
        g = jnp.concatenate(rows, axis=0)
        logits = lax.dot_general(g, wg_ref[...], _CONTRACT_LAST,
                                 preferred_element_type=jnp.float32) + bg_ref[...]
        pad = jnp.full((B, 16 - E), -1e30, jnp.float32)
        logits_ref[...] = jnp.concatenate([logits, pad], axis=1)


def _sc_gate_body(logits_hbm, w_hbm, buf, out_buf):
    c = lax.axis_index("c")
    s_ = lax.axis_index("s")

    idx = lax.iota(jnp.int32, 16)

    gdn = lax.GatherDimensionNumbers(offset_dims=(), collapsed_slice_dims=(0,),
                                     start_index_map=(0,))

    def lane_gather(vec, perm):
        return lax.gather(vec, perm.reshape(16, 1), gdn, (1,),
                          mode=lax.GatherScatterMode.PROMISE_IN_BOUNDS)

    def allred(vec, op):
        for sh in (8, 4, 2, 1):
            perm = jnp.bitwise_xor(idx, sh)
            vec = op(vec, lane_gather(vec, perm))
        return vec

    @pl.when(jnp.logical_and(c == 0, s_ == 0))
    def _():
        pltpu.sync_copy(logits_hbm, buf)
        for b in range(B):
            x = buf[b]
            m = allred(x, jnp.maximum)
            p = jnp.exp(x - m)
            p = p / allred(p, jnp.add)
            v1 = allred(p, jnp.maximum)
            i1 = allred(jnp.where(p >= v1, idx, 16), jnp.minimum)
            mask1 = idx == i1
            p2 = jnp.where(mask1, -1.0, p)
            v2 = allred(p2, jnp.maximum)
            i2 = allred(jnp.where(p2 >= v2, idx, 16), jnp.minimum)
            out_buf[b] = (jnp.where(mask1, v1, 0.0)
                          + jnp.where(idx == i2, v2, 0.0))
        pltpu.sync_copy(out_buf, w_hbm)


def _combine_body(w_ref, we_ref, be_ref, wc_ref, bc_ref):
    we = we_ref[...].astype(jnp.float32)
    for b in range(B):
        acc = w_ref[b, 0] * we[0]
        for e in range(1, E):
            acc = acc + w_ref[b, e] * we[e]
        wc_ref[b] = acc.astype(jnp.bfloat16)

    @pl.when(pl.program_id(0) == 0)
    def _():
        be = be_ref[...]
        for b in range(B):
            bcc = w_ref[b, 0] * be[0:1]
            for e in range(1, E):
                bcc = bcc + w_ref[b, e] * be[e:e + 1]
            bc_ref[b] = bcc


def _moe_out_body(ctx_ref, wc_ref, bc_ref, wo_ref, bo_ref, out_ref):
    HS = SB // 2
    for half in range(2):
        sl = slice(half * HS, (half + 1) * HS)
        ctx = jnp.concatenate([ctx_ref[0, h, sl] for h in range(H)],
                              axis=1)
        moe = lax.dot_general(ctx, wc_ref[0], _CONTRACT_LAST,
                              preferred_element_type=jnp.float32) + bc_ref[0]
        out = lax.dot_general(moe.astype(jnp.bfloat16), wo_ref[...],
                              _CONTRACT_LAST,
                              preferred_element_type=jnp.float32) + bo_ref[...]
        out_ref[0, sl] = out


def kernel(hidden_states, Wq, bq, Wk, bk, Wv, bv, We, be, Wg, bg, Wo, bo):
    scale = math.log2(math.e) / math.sqrt(DH)
    wq16 = (Wq * scale).astype(jnp.bfloat16)
    wk16 = Wk.astype(jnp.bfloat16)
    wv16 = Wv.astype(jnp.bfloat16)
    we16 = We.astype(jnp.bfloat16)
    wo16 = Wo.astype(jnp.bfloat16)
    bq2 = (bq * scale).reshape(1, D)
    bk2 = bk.reshape(1, D)
    bv2 = bv.reshape(1, D)
    bg2 = bg.reshape(1, E)
    bo2 = bo.reshape(1, D)

    n_sb = S // SBLK
    qkv_struct = jax.ShapeDtypeStruct((B, H, S, DH), jnp.bfloat16)
    q, k, v = pl.pallas_call(
        _qkv_body,
        grid=(B, n_sb),
        in_specs=[
            pl.BlockSpec((1, SBLK, D), lambda b, si: (b, si, 0)),
            pl.BlockSpec((D, D), lambda b, si: (0, 0)),
            pl.BlockSpec((1, D), lambda b, si: (0, 0)),
            pl.BlockSpec((D, D), lambda b, si: (0, 0)),
            pl.BlockSpec((1, D), lambda b, si: (0, 0)),
            pl.BlockSpec((D, D), lambda b, si: (0, 0)),
            pl.BlockSpec((1, D), lambda b, si: (0, 0)),
        ],
        out_specs=[
            pl.BlockSpec((1, H, SBLK, DH), lambda b, si: (b, 0, si, 0)),
            pl.BlockSpec((1, H, SBLK, DH), lambda b, si: (b, 0, si, 0)),
            pl.BlockSpec((1, H, SBLK, DH), lambda b, si: (b, 0, si, 0)),
        ],
        out_shape=[qkv_struct] * 3,
    )(hidden_states, wq16, bq2, wk16, bk2, wv16, bv2)

    nq = S // SQ
    ctx4, _means, logits16 = pl.pallas_call(
        _attn_body,
        grid=(B, H, nq),
        in_specs=[
            pl.BlockSpec((1, 1, SQ, DH), lambda b, h, qi: (b, h, qi, 0)),
            pl.BlockSpec((1, 1, S, DH), lambda b, h, qi: (b, h, 0, 0)),
            pl.BlockSpec((1, 1, S, DH), lambda b, h, qi: (b, h, 0, 0)),
            pl.BlockSpec((E, D), lambda b, h, qi: (0, 0)),
            pl.BlockSpec((1, E), lambda b, h, qi: (0, 0)),
        ],
        out_specs=[
            pl.BlockSpec((1, 1, SQ, DH), lambda b, h, qi: (b, h, qi, 0)),
            pl.BlockSpec((B, H, 8, DH), lambda b, h, qi: (0, 0, 0, 0)),
            pl.BlockSpec((B, 16), lambda b, h, qi: (0, 0)),
        ],
        out_shape=[
            jax.ShapeDtypeStruct((B, H, S, DH), jnp.bfloat16),
            jax.ShapeDtypeStruct((B, H, 8, DH), jnp.float32),
            jax.ShapeDtypeStruct((B, 16), jnp.float32),
        ],
    )(q, k, v, Wg, bg2)

    sc_gate = pl.kernel(
        _sc_gate_body,
        out_type=jax.ShapeDtypeStruct((B, 16), jnp.float32),
        mesh=plsc.VectorSubcoreMesh(core_axis_name="c", subcore_axis_name="s"),
        scratch_types=[
            pltpu.VMEM((B, 16), jnp.float32),
            pltpu.VMEM((B, 16), jnp.float32),
        ],
    )
    w = sc_gate(logits16)[:, :E]

    n_rb = D // RB
    wc, bc3 = pl.pallas_call(
        _combine_body,
        grid=(n_rb,),
        in_specs=[
            pl.BlockSpec(memory_space=pltpu.SMEM),
            pl.BlockSpec((E, RB, D), lambda i: (0, i, 0)),
            pl.BlockSpec((E, D), lambda i: (0, 0)),
        ],
        out_specs=[
            pl.BlockSpec((B, RB, D), lambda i: (0, i, 0)),
            pl.BlockSpec((B, 1, D), lambda i: (0, 0, 0)),
        ],
        out_shape=[
            jax.ShapeDtypeStruct((B, D, D), jnp.bfloat16),
            jax.ShapeDtypeStruct((B, 1, D), jnp.float32),
        ],
    )(w, we16, be)
    ns = S // SB
    out = pl.pallas_call(
        _moe_out_body,
        grid=(B, ns),
        in_specs=[
            pl.BlockSpec((1, H, SB, DH), lambda b, si: (b, 0, si, 0)),
            pl.BlockSpec((1, D, D), lambda b, si: (b, 0, 0)),
            pl.BlockSpec((1, 1, D), lambda b, si: (b, 0, 0)),
            pl.BlockSpec((D, D), lambda b, si: (0, 0)),
            pl.BlockSpec((1, D), lambda b, si: (0, 0)),
        ],
        out_specs=pl.BlockSpec((1, SB, D), lambda b, si: (b, si, 0)),
        out_shape=jax.ShapeDtypeStruct((B, S, D), jnp.float32),
    )(ctx4, wc, bc3, wo16, bo2)

    return out


# speedup vs baseline: 1.1667x; 1.0163x over previous
import functools
import math

import jax
import jax.numpy as jnp
from jax import lax
from jax.experimental import pallas as pl
from jax.experimental.pallas import tpu as pltpu
from jax.experimental.pallas import tpu_sc as plsc

B, S, D = 2, 2048, 1024
H = 16
E = 8
DH = D // H

SBLK = 1024
SQ = 2048
NCH = 4
HQ = SQ // NCH
RB = 256
SB = 1024

_CONTRACT_LAST = (((1,), (1,)), ((), ()))
_CONTRACT_STD = (((1,), (0,)), ((), ()))


def _qkv_body(x_ref, wq_ref, bq_ref, wk_ref, bk_ref, wv_ref, bv_ref,
              q_ref, k_ref, v_ref):
    x = x_ref[0].astype(jnp.bfloat16)
    q = (lax.dot_general(x, wq_ref[...], _CONTRACT_LAST,
                         preferred_element_type=jnp.float32)
         + bq_ref[...]).astype(jnp.bfloat16)
    k = (lax.dot_general(x, wk_ref[...], _CONTRACT_LAST,
                         preferred_element_type=jnp.float32)
         + bk_ref[...]).astype(jnp.bfloat16)
    v = (lax.dot_general(x, wv_ref[...], _CONTRACT_LAST,
                         preferred_element_type=jnp.float32)
         + bv_ref[...]).astype(jnp.bfloat16)
    for h in range(H):
        sl = slice(h * DH, (h + 1) * DH)
        q_ref[0, h] = q[:, sl]
        k_ref[0, h] = k[:, sl]
        v_ref[0, h] = v[:, sl]


def _attn_body(q_ref, k_ref, v_ref, wg_ref, bg_ref, ctx_ref, mean_ref,
               logits_ref):
    b_ = pl.program_id(0)
    h_ = pl.program_id(1)
    qi = pl.program_id(2)

    k = k_ref[0, 0]
    v = v_ref[0, 0]
    v_aug = jnp.concatenate([v, jnp.ones((S, 1), jnp.bfloat16)], axis=1)

    part = jnp.zeros((1, DH), jnp.float32)
    for half in range(NCH):
        q = q_ref[0, 0, half * HQ:(half + 1) * HQ]
        scores = lax.dot_general(q, k, _CONTRACT_LAST,
                                 preferred_element_type=jnp.float32)
        m = jnp.max(scores, axis=1, keepdims=True)
        p = jnp.exp2(scores - m)
        ctx_aug = lax.dot_general(p.astype(jnp.bfloat16), v_aug, _CONTRACT_STD,
                                  preferred_element_type=jnp.float32)
        ctx = ctx_aug[:, :DH] / ctx_aug[:, DH:DH + 1]
        ctx_ref[0, 0, half * HQ:(half + 1) * HQ] = ctx.astype(jnp.bfloat16)
        part = part + jnp.sum(ctx, axis=0, keepdims=True)

    @pl.when(jnp.logical_and(b_ == 0, jnp.logical_and(h_ == 0, qi == 0)))
    def _():
        mean_ref[...] = jnp.zeros_like(mean_ref)

    mean_ref[pl.ds(b_, 1), pl.ds(h_, 1), 0:1, :] += part.reshape(
        1, 1, 1, DH) * (1.0 / S)

    is_last = jnp.logical_and(
        b_ == B - 1, jnp.logical_and(h_ == H - 1, qi == pl.num_programs(2) - 1))

    @pl.when(is_last)
    def _():
        rows = []
        for b in range(B):
            rows.append(jnp.concatenate(
                [mean_ref[b, h, 0:1, :] for h in range(H)], axis=1))
        g = jnp.concatenate(rows, axis=0)
        logits = lax.dot_general(g, wg_ref[...], _CONTRACT_LAST,
                                 preferred_element_type=jnp.float32) + bg_ref[...]
        pad = jnp.full((B, 16 - E), -1e30, jnp.float32)
        logits_ref[...] = jnp.concatenate([logits, pad], axis=1)


def _sc_gate_body(logits_hbm, w_hbm, buf, out_buf):
    c = lax.axis_index("c")
    s_ = lax.axis_index("s")

    idx = lax.iota(jnp.int32, 16)

    gdn = lax.GatherDimensionNumbers(offset_dims=(), collapsed_slice_dims=(0,),
                                     start_index_map=(0,))

    def lane_gather(vec, perm):
        return lax.gather(vec, perm.reshape(16, 1), gdn, (1,),
                          mode=lax.GatherScatterMode.PROMISE_IN_BOUNDS)

    def allred(vec, op):
        for sh in (8, 4, 2, 1):
            perm = jnp.bitwise_xor(idx, sh)
            vec = op(vec, lane_gather(vec, perm))
        return vec

    @pl.when(jnp.logical_and(c == 0, s_ == 0))
    def _():
        pltpu.sync_copy(logits_hbm, buf)
        for b in range(B):
            x = buf[b]
            m = allred(x, jnp.maximum)
            p = jnp.exp(x - m)
            p = p / allred(p, jnp.add)
            v1 = allred(p, jnp.maximum)
            i1 = allred(jnp.where(p >= v1, idx, 16), jnp.minimum)
            mask1 = idx == i1
            p2 = jnp.where(mask1, -1.0, p)
            v2 = allred(p2, jnp.maximum)
            i2 = allred(jnp.where(p2 >= v2, idx, 16), jnp.minimum)
            out_buf[b] = (jnp.where(mask1, v1, 0.0)
                          + jnp.where(idx == i2, v2, 0.0))
        pltpu.sync_copy(out_buf, w_hbm)


def _moe_out_body(ctx_ref, w_ref, we_ref, be_ref, wo_ref, bo_ref, out_ref,
                  wc_s, bc_s):
    b_ = pl.program_id(0)
    si = pl.program_id(1)

    @pl.when(si == 0)
    def _():
        for rb in range(D // RB):
            sl = slice(rb * RB, (rb + 1) * RB)
            acc = w_ref[b_, 0] * we_ref[0, sl].astype(jnp.float32)
            for e in range(1, E):
                acc = acc + w_ref[b_, e] * we_ref[e, sl].astype(jnp.float32)
            wc_s[sl] = acc.astype(jnp.bfloat16)
        bcc = w_ref[b_, 0] * be_ref[0:1]
        for e in range(1, E):
            bcc = bcc + w_ref[b_, e] * be_ref[e:e + 1]
        bc_s[...] = bcc

    wc = wc_s[...]
    bc = bc_s[...]
    HS = SB // 2
    for half in range(2):
        sl = slice(half * HS, (half + 1) * HS)
        ctx = jnp.concatenate([ctx_ref[0, h, sl] for h in range(H)],
                              axis=1)
        moe = lax.dot_general(ctx, wc, _CONTRACT_LAST,
                              preferred_element_type=jnp.float32) + bc
        out = lax.dot_general(moe.astype(jnp.bfloat16), wo_ref[...],
                              _CONTRACT_LAST,
                              preferred_element_type=jnp.float32) + bo_ref[...]
        out_ref[0, sl] = out


def kernel(hidden_states, Wq, bq, Wk, bk, Wv, bv, We, be, Wg, bg, Wo, bo):
    scale = math.log2(math.e) / math.sqrt(DH)
    wq16 = (Wq * scale).astype(jnp.bfloat16)
    wk16 = Wk.astype(jnp.bfloat16)
    wv16 = Wv.astype(jnp.bfloat16)
    we16 = We.astype(jnp.bfloat16)
    wo16 = Wo.astype(jnp.bfloat16)
    bq2 = (bq * scale).reshape(1, D)
    bk2 = bk.reshape(1, D)
    bv2 = bv.reshape(1, D)
    bg2 = bg.reshape(1, E)
    bo2 = bo.reshape(1, D)

    n_sb = S // SBLK
    qkv_struct = jax.ShapeDtypeStruct((B, H, S, DH), jnp.bfloat16)
    q, k, v = pl.pallas_call(
        _qkv_body,
        grid=(B, n_sb),
        in_specs=[
            pl.BlockSpec((1, SBLK, D), lambda b, si: (b, si, 0)),
            pl.BlockSpec((D, D), lambda b, si: (0, 0)),
            pl.BlockSpec((1, D), lambda b, si: (0, 0)),
            pl.BlockSpec((D, D), lambda b, si: (0, 0)),
            pl.BlockSpec((1, D), lambda b, si: (0, 0)),
            pl.BlockSpec((D, D), lambda b, si: (0, 0)),
            pl.BlockSpec((1, D), lambda b, si: (0, 0)),
        ],
        out_specs=[
            pl.BlockSpec((1, H, SBLK, DH), lambda b, si: (b, 0, si, 0)),
            pl.BlockSpec((1, H, SBLK, DH), lambda b, si: (b, 0, si, 0)),
            pl.BlockSpec((1, H, SBLK, DH), lambda b, si: (b, 0, si, 0)),
        ],
        out_shape=[qkv_struct] * 3,
    )(hidden_states, wq16, bq2, wk16, bk2, wv16, bv2)

    nq = S // SQ
    ctx4, _means, logits16 = pl.pallas_call(
        _attn_body,
        grid=(B, H, nq),
        in_specs=[
            pl.BlockSpec((1, 1, SQ, DH), lambda b, h, qi: (b, h, qi, 0)),
            pl.BlockSpec((1, 1, S, DH), lambda b, h, qi: (b, h, 0, 0)),
            pl.BlockSpec((1, 1, S, DH), lambda b, h, qi: (b, h, 0, 0)),
            pl.BlockSpec((E, D), lambda b, h, qi: (0, 0)),
            pl.BlockSpec((1, E), lambda b, h, qi: (0, 0)),
        ],
        out_specs=[
            pl.BlockSpec((1, 1, SQ, DH), lambda b, h, qi: (b, h, qi, 0)),
            pl.BlockSpec((B, H, 8, DH), lambda b, h, qi: (0, 0, 0, 0)),
            pl.BlockSpec((B, 16), lambda b, h, qi: (0, 0)),
        ],
        out_shape=[
            jax.ShapeDtypeStruct((B, H, S, DH), jnp.bfloat16),
            jax.ShapeDtypeStruct((B, H, 8, DH), jnp.float32),
            jax.ShapeDtypeStruct((B, 16), jnp.float32),
        ],
    )(q, k, v, Wg, bg2)

    sc_gate = pl.kernel(
        _sc_gate_body,
        out_type=jax.ShapeDtypeStruct((B, 16), jnp.float32),
        mesh=plsc.VectorSubcoreMesh(core_axis_name="c", subcore_axis_name="s"),
        scratch_types=[
            pltpu.VMEM((B, 16), jnp.float32),
            pltpu.VMEM((B, 16), jnp.float32),
        ],
    )
    w = sc_gate(logits16)[:, :E]

    ns = S // SB
    out = pl.pallas_call(
        _moe_out_body,
        grid=(B, ns),
        in_specs=[
            pl.BlockSpec((1, H, SB, DH), lambda b, si: (b, 0, si, 0)),
            pl.BlockSpec(memory_space=pltpu.SMEM),
            pl.BlockSpec((E, D, D), lambda b, si: (0, 0, 0)),
            pl.BlockSpec((E, D), lambda b, si: (0, 0)),
            pl.BlockSpec((D, D), lambda b, si: (0, 0)),
            pl.BlockSpec((1, D), lambda b, si: (0, 0)),
        ],
        out_specs=pl.BlockSpec((1, SB, D), lambda b, si: (b, si, 0)),
        out_shape=jax.ShapeDtypeStruct((B, S, D), jnp.float32),
        scratch_shapes=[
            pltpu.VMEM((D, D), jnp.bfloat16),
            pltpu.VMEM((1, D), jnp.float32),
        ],
    )(ctx4, w, we16, be, wo16, bo2)

    return out


# speedup vs baseline: 1.1675x; 1.0007x over previous
import functools
import math

import jax
import jax.numpy as jnp
from jax import lax
from jax.experimental import pallas as pl
from jax.experimental.pallas import tpu as pltpu
from jax.experimental.pallas import tpu_sc as plsc

B, S, D = 2, 2048, 1024
H = 16
E = 8
DH = D // H

SBLK = 1024
SQ = 2048
NCH = 4
HQ = SQ // NCH
RB = 256
SB = 1024

_CONTRACT_LAST = (((1,), (1,)), ((), ()))
_CONTRACT_STD = (((1,), (0,)), ((), ()))


def _qkv_attn_body(x_ref, wq_ref, bq_ref, wk_ref, bk_ref, wv_ref, bv_ref,
                   wg_ref, bg_ref, ctx_ref, mean_ref, logits_ref,
                   qs, ks, vs):
    b_ = pl.program_id(0)
    ph = pl.program_id(1)

    @pl.when(ph == 0)
    def _():
        for rw in range(S // SBLK):
            rsl = slice(rw * SBLK, (rw + 1) * SBLK)
            x = x_ref[0, rsl].astype(jnp.bfloat16)
            q = (lax.dot_general(x, wq_ref[...], _CONTRACT_LAST,
                                 preferred_element_type=jnp.float32)
                 + bq_ref[...]).astype(jnp.bfloat16)
            k = (lax.dot_general(x, wk_ref[...], _CONTRACT_LAST,
                                 preferred_element_type=jnp.float32)
                 + bk_ref[...]).astype(jnp.bfloat16)
            v = (lax.dot_general(x, wv_ref[...], _CONTRACT_LAST,
                                 preferred_element_type=jnp.float32)
                 + bv_ref[...]).astype(jnp.bfloat16)
            for h in range(H):
                csl = slice(h * DH, (h + 1) * DH)
                qs[h, rsl] = q[:, csl]
                ks[h, rsl] = k[:, csl]
                vs[h, rsl] = v[:, csl]

    @pl.when(jnp.logical_and(b_ == 0, ph == 0))
    def _():
        mean_ref[...] = jnp.zeros_like(mean_ref)

    @pl.when(ph > 0)
    def _():
        h_ = ph - 1
        k = ks[pl.ds(h_, 1)][0]
        v = vs[pl.ds(h_, 1)][0]
        v_aug = jnp.concatenate([v, jnp.ones((S, 1), jnp.bfloat16)], axis=1)

        part = jnp.zeros((1, DH), jnp.float32)
        for half in range(NCH):
            rsl = pl.ds(half * HQ, HQ)
            q = qs[pl.ds(h_, 1), rsl][0]
            scores = lax.dot_general(q, k, _CONTRACT_LAST,
                                     preferred_element_type=jnp.float32)
            m = jnp.max(scores, axis=1, keepdims=True)
            p = jnp.exp2(scores - m)
            ctx_aug = lax.dot_general(p.astype(jnp.bfloat16), v_aug,
                                      _CONTRACT_STD,
                                      preferred_element_type=jnp.float32)
            ctx = ctx_aug[:, :DH] / ctx_aug[:, DH:DH + 1]
            ctx_ref[0, 0, half * HQ:(half + 1) * HQ] = ctx.astype(jnp.bfloat16)
            part = part + jnp.sum(ctx, axis=0, keepdims=True)

        mean_ref[pl.ds(b_, 1), pl.ds(h_, 1), 0:1, :] += part.reshape(
            1, 1, 1, DH) * (1.0 / S)

        is_last = jnp.logical_and(b_ == B - 1, ph == H)

        @pl.when(is_last)
        def _():
            rows = []
            for b in range(B):
                rows.append(jnp.concatenate(
                    [mean_ref[b, h, 0:1, :] for h in range(H)], axis=1))
            g = jnp.concatenate(rows, axis=0)
            logits = lax.dot_general(g, wg_ref[...], _CONTRACT_LAST,
                                     preferred_element_type=jnp.float32
                                     ) + bg_ref[...]
            pad = jnp.full((B, 16 - E), -1e30, jnp.float32)
            logits_ref[...] = jnp.concatenate([logits, pad], axis=1)


def _sc_gate_body(logits_hbm, w_hbm, buf, out_buf):
    c = lax.axis_index("c")
    s_ = lax.axis_index("s")

    idx = lax.iota(jnp.int32, 16)

    gdn = lax.GatherDimensionNumbers(offset_dims=(), collapsed_slice_dims=(0,),
                                     start_index_map=(0,))

    def lane_gather(vec, perm):
        return lax.gather(vec, perm.reshape(16, 1), gdn, (1,),
                          mode=lax.GatherScatterMode.PROMISE_IN_BOUNDS)

    def allred(vec, op):
        for sh in (8, 4, 2, 1):
            perm = jnp.bitwise_xor(idx, sh)
            vec = op(vec, lane_gather(vec, perm))
        return vec

    @pl.when(jnp.logical_and(c == 0, s_ == 0))
    def _():
        pltpu.sync_copy(logits_hbm, buf)
        for b in range(B):
            x = buf[b]
            m = allred(x, jnp.maximum)
            p = jnp.exp(x - m)
            p = p / allred(p, jnp.add)
            v1 = allred(p, jnp.maximum)
            i1 = allred(jnp.where(p >= v1, idx, 16), jnp.minimum)
            mask1 = idx == i1
            p2 = jnp.where(mask1, -1.0, p)
            v2 = allred(p2, jnp.maximum)
            i2 = allred(jnp.where(p2 >= v2, idx, 16), jnp.minimum)
            out_buf[b] = (jnp.where(mask1, v1, 0.0)
                          + jnp.where(idx == i2, v2, 0.0))
        pltpu.sync_copy(out_buf, w_hbm)


def _moe_out_body(ctx_ref, w_ref, we_ref, be_ref, wo_ref, bo_ref, out_ref,
                  wc_s, bc_s):
    b_ = pl.program_id(0)
    si = pl.program_id(1)

    @pl.when(si == 0)
    def _():
        for rb in range(D // RB):
            sl = slice(rb * RB, (rb + 1) * RB)
            acc = w_ref[b_, 0] * we_ref[0, sl].astype(jnp.float32)
            for e in range(1, E):
                acc = acc + w_ref[b_, e] * we_ref[e, sl].astype(jnp.float32)
            wc_s[sl] = acc.astype(jnp.bfloat16)
        bcc = w_ref[b_, 0] * be_ref[0:1]
        for e in range(1, E):
            bcc = bcc + w_ref[b_, e] * be_ref[e:e + 1]
        bc_s[...] = bcc

    wc = wc_s[...]
    bc = bc_s[...]
    HS = SB // 2
    for half in range(2):
        sl = slice(half * HS, (half + 1) * HS)
        ctx = jnp.concatenate([ctx_ref[0, h, sl] for h in range(H)],
                              axis=1)
        moe = lax.dot_general(ctx, wc, _CONTRACT_LAST,
                              preferred_element_type=jnp.float32) + bc
        out = lax.dot_general(moe.astype(jnp.bfloat16), wo_ref[...],
                              _CONTRACT_LAST,
                              preferred_element_type=jnp.float32) + bo_ref[...]
        out_ref[0, sl] = out


def kernel(hidden_states, Wq, bq, Wk, bk, Wv, bv, We, be, Wg, bg, Wo, bo):
    scale = math.log2(math.e) / math.sqrt(DH)
    wq16 = (Wq * scale).astype(jnp.bfloat16)
    wk16 = Wk.astype(jnp.bfloat16)
    wv16 = Wv.astype(jnp.bfloat16)
    we16 = We.astype(jnp.bfloat16)
    wo16 = Wo.astype(jnp.bfloat16)
    bq2 = (bq * scale).reshape(1, D)
    bk2 = bk.reshape(1, D)
    bv2 = bv.reshape(1, D)
    bg2 = bg.reshape(1, E)
    bo2 = bo.reshape(1, D)

    ctx4, _means, logits16 = pl.pallas_call(
        _qkv_attn_body,
        grid=(B, 1 + H),
        in_specs=[
            pl.BlockSpec((1, S, D), lambda b, ph: (b, 0, 0)),
            pl.BlockSpec((D, D), lambda b, ph: (0, 0)),
            pl.BlockSpec((1, D), lambda b, ph: (0, 0)),
            pl.BlockSpec((D, D), lambda b, ph: (0, 0)),
            pl.BlockSpec((1, D), lambda b, ph: (0, 0)),
            pl.BlockSpec((D, D), lambda b, ph: (0, 0)),
            pl.BlockSpec((1, D), lambda b, ph: (0, 0)),
            pl.BlockSpec((E, D), lambda b, ph: (0, 0)),
            pl.BlockSpec((1, E), lambda b, ph: (0, 0)),
        ],
        out_specs=[
            pl.BlockSpec((1, 1, S, DH),
                         lambda b, ph: (b, jnp.maximum(ph - 1, 0), 0, 0)),
            pl.BlockSpec((B, H, 8, DH), lambda b, ph: (0, 0, 0, 0)),
            pl.BlockSpec((B, 16), lambda b, ph: (0, 0)),
        ],
        out_shape=[
            jax.ShapeDtypeStruct((B, H, S, DH), jnp.bfloat16),
            jax.ShapeDtypeStruct((B, H, 8, DH), jnp.float32),
            jax.ShapeDtypeStruct((B, 16), jnp.float32),
        ],
        scratch_shapes=[
            pltpu.VMEM((H, S, DH), jnp.bfloat16),
            pltpu.VMEM((H, S, DH), jnp.bfloat16),
            pltpu.VMEM((H, S, DH), jnp.bfloat16),
        ],
    )(hidden_states, wq16, bq2, wk16, bk2, wv16, bv2, Wg, bg2)

    sc_gate = pl.kernel(
        _sc_gate_body,
        out_type=jax.ShapeDtypeStruct((B, 16), jnp.float32),
        mesh=plsc.VectorSubcoreMesh(core_axis_name="c", subcore_axis_name="s"),
        scratch_types=[
            pltpu.VMEM((B, 16), jnp.float32),
            pltpu.VMEM((B, 16), jnp.float32),
        ],
    )
    w = sc_gate(logits16)[:, :E]

    ns = S // SB
    out = pl.pallas_call(
        _moe_out_body,
        grid=(B, ns),
        in_specs=[
            pl.BlockSpec((1, H, SB, DH), lambda b, si: (b, 0, si, 0)),
            pl.BlockSpec(memory_space=pltpu.SMEM),
            pl.BlockSpec((E, D, D), lambda b, si: (0, 0, 0)),
            pl.BlockSpec((E, D), lambda b, si: (0, 0)),
            pl.BlockSpec((D, D), lambda b, si: (0, 0)),
            pl.BlockSpec((1, D), lambda b, si: (0, 0)),
        ],
        out_specs=pl.BlockSpec((1, SB, D), lambda b, si: (b, si, 0)),
        out_shape=jax.ShapeDtypeStruct((B, S, D), jnp.float32),
        scratch_shapes=[
            pltpu.VMEM((D, D), jnp.bfloat16),
            pltpu.VMEM((1, D), jnp.float32),
        ],
    )(ctx4, w, we16, be, wo16, bo2)

    return out
